# baseline (speedup 1.0000x reference)
import jax
import jax.numpy as jnp
from jax import lax
from jax.experimental import pallas as pl
from jax.experimental.pallas import tpu as pltpu
from jax.experimental.pallas import tpu_sc as plsc

N = 512 * 512
K_SEL = 1024
NS = 16
CH = N // NS
NV = CH // 16
MSB = -0x80000000
IMAX = 0x7FFFFFFF

_mesh = plsc.VectorSubcoreMesh(
    core_axis_name="c", subcore_axis_name="s", num_cores=1, num_subcores=NS)


def _splat(v, dtype=jnp.int32):
    return jnp.full((16,), v, dtype=dtype)


_SCRATCH = [
    pltpu.VMEM((CH,), jnp.float32),
    pltpu.VMEM((CH + 16,), jnp.float32),
    pltpu.VMEM((CH + 16,), jnp.float32),
    pltpu.VMEM((64,), jnp.int32),
    pltpu.VMEM((NS * 64,), jnp.int32),
    pltpu.VMEM((16,), jnp.float32),
    pltpu.VMEM((32,), jnp.int32),
    pltpu.VMEM_SHARED((2 * NS * 32,), jnp.int32),
    pltpu.VMEM_SHARED((NS * 64,), jnp.int32),
]


def _toploss_body(x_hbm, out_hbm, xv, bT, bB,
                  stage_i, gbuf_i, outv, nbuf, sh_i, sh_w):
    cid = lax.axis_index("c")
    sid = lax.axis_index("s")
    base = sid * CH
    iota = lax.iota(jnp.int32, 16)
    zero_i = _splat(0)
    zero_f = _splat(0.0, jnp.float32)

    def popcnt(m):
        return plsc.all_reduce_population_count(m)

    def exchange_i32(lo_vec, hi_vec, parity, red):
        stage_i[pl.ds(0, 16)] = lo_vec
        stage_i[pl.ds(16, 16)] = hi_vec
        pltpu.sync_copy(stage_i.at[pl.ds(0, 32)],
                        sh_i.at[pl.ds(parity * (NS * 32) + sid * 32, 32)])
        plsc.subcore_barrier()
        pltpu.sync_copy(sh_i.at[pl.ds(parity * (NS * 32), NS * 32)],
                        gbuf_i.at[pl.ds(0, NS * 32)])
        lo = gbuf_i[pl.ds(0, 16)]
        hi = gbuf_i[pl.ds(16, 16)]
        for w in range(1, NS):
            lo = red(lo, gbuf_i[pl.ds(w * 32, 16)])
            hi = red(hi, gbuf_i[pl.ds(w * 32 + 16, 16)])
        return lo, hi

    def skey_of(xvec):
        b = plsc.bitcast(xvec, jnp.int32)
        return jnp.where(b >= 0, b, MSB - b)

    def inv_val(kvv):
        bits = jnp.where(kvv >= 0, kvv, MSB - kvv)
        return plsc.bitcast(bits, jnp.float32)

    pltpu.sync_copy(x_hbm.at[pl.ds(base, CH)], xv)

    @plsc.parallel_loop(0, NV, unroll=8,
                        carry=(_splat(-jnp.inf, jnp.float32),
                               _splat(jnp.inf, jnp.float32)))
    def _minmax(i, carry):
        mxf, mnf = carry
        xvec = xv[pl.ds(i * 16, 16)]
        return jnp.maximum(mxf, xvec), jnp.minimum(mnf, xvec)

    mxf, mnf = _minmax
    mxg, mng_inv = exchange_i32(skey_of(mxf), ~skey_of(mnf), 0, jnp.maximum)
    mng = ~mng_inv
    kmax_v = _splat(jnp.max(mxg))
    kmin_v = _splat(jnp.min(mng))
    x_max = inv_val(kmax_v)
    x_min = inv_val(kmin_v)

    rng_v = (x_max - x_min) * (1.0 / 160.0)
    tau_t_try = x_max - rng_v
    tau_b_try = x_min + rng_v

    def compact_pass(tt, tb):
        @plsc.parallel_loop(0, NV, unroll=8,
                            carry=(_splat(-1), _splat(-1)))
        def _comp(i, carry):
            offTm1, offBm1 = carry
            xvec = xv[pl.ds(i * 16, 16)]
            mT = xvec >= tt
            mB = xvec <= tb
            plsc.store_scatter(
                bT, [offTm1 + plsc.cumsum(mT.astype(jnp.int32))],
                xvec, mask=mT)
            plsc.store_scatter(
                bB, [offBm1 + plsc.cumsum(mB.astype(jnp.int32))],
                xvec, mask=mB)
            return offTm1 + popcnt(mT), offBm1 + popcnt(mB)

        om1T, om1B = _comp
        return om1T + 1, om1B + 1

    offT, offB = compact_pass(tau_t_try, tau_b_try)
    cT, cB = exchange_i32(offT, offB, 1, jnp.add)
    okT = cT >= K_SEL
    okB = cB >= K_SEL
    tau_t = jnp.where(okT, tau_t_try, x_min)
    tau_b = jnp.where(okB, tau_b_try, x_max)
    nbuf[pl.ds(0, 16)] = offT
    nbuf[pl.ds(16, 16)] = offB
    redo = jnp.max(jnp.where(okT & okB, zero_i, _splat(1)))

    @pl.when(redo > 0)
    def _():
        oT, oB = compact_pass(tau_t, tau_b)
        nbuf[pl.ds(0, 16)] = oT
        nbuf[pl.ds(16, 16)] = oB

    nT_v = nbuf[pl.ds(0, 16)]
    nB_v = nbuf[pl.ds(16, 16)]
    tripT = (jnp.max(nT_v) + 15) // 16
    tripB = (jnp.max(nB_v) + 15) // 16
    plsc.store_scatter(bT, [nT_v + iota], _splat(-jnp.inf, jnp.float32))
    plsc.store_scatter(bB, [nB_v + iota], _splat(jnp.inf, jnp.float32))
    tripJ = jnp.maximum(tripT, tripB)

    def padT(i, c):
        bT[pl.ds((tripT + i) * 16, 16)] = _splat(-jnp.inf, jnp.float32)
        return c

    def padB(i, c):
        bB[pl.ds((tripB + i) * 16, 16)] = _splat(jnp.inf, jnp.float32)
        return c

    lax.fori_loop(0, tripJ - tripT, padT, 0)
    lax.fori_loop(0, tripJ - tripB, padB, 0)

    ktau_t = skey_of(tau_t)
    ktau_b = skey_of(tau_b)

    def ehigh(diff):
        f = diff.astype(jnp.float32)
        e = lax.shift_right_logical(plsc.bitcast(f, jnp.int32), 23) - 127
        e = jnp.where(diff == 0, _splat(-1000), e)
        return jnp.where(diff < 0, _splat(31), e)

    e_joint = jnp.maximum(ehigh(ktau_t ^ kmax_v), ehigh(ktau_b ^ kmin_v))
    g0_v = jnp.clip((_splat(31) - e_joint) // 4, 0, 8)
    nclear = _splat(32) - 4 * g0_v
    lowmask = jnp.where(g0_v == 0, _splat(-1),
                        lax.shift_left(_splat(1), nclear) - 1)
    ut0 = ((kmax_v ^ MSB) & ~lowmask)
    ub0 = ((kmin_v ^ MSB) & ~lowmask)
    g0 = jnp.max(g0_v)

    def group(g, carry):
        ut, ub = carry
        s = 28 - 4 * g
        low = lax.shift_left(jnp.int32(1), s) - 1
        cand_ts = [inv_val((ut | lax.shift_left(_splat(j), s)) ^ MSB)
                   for j in range(1, 16)]
        cand_bs = [inv_val((ub | lax.shift_left(_splat(j), s) | low) ^ MSB)
                   for j in range(15)]

        def cnt_TB(i, accs):
            aT, aB = accs
            xtv = bT[pl.ds(i * 16, 16)]
            xbv = bB[pl.ds(i * 16, 16)]
            aT = tuple(a + popcnt(xtv >= c) for a, c in zip(aT, cand_ts))
            aB = tuple(a + popcnt(xbv <= c) for a, c in zip(aB, cand_bs))
            return aT, aB

        accT, accB = lax.fori_loop(
            0, tripJ, cnt_TB, ((zero_i,) * 15, (zero_i,) * 15))
        packT = zero_i
        for j, a in enumerate(accT):
            packT = packT + jnp.where(iota == j + 1, a, zero_i)
        packB = zero_i
        for j, a in enumerate(accB):
            packB = packB + jnp.where(iota == j, a, zero_i)
        totT, totB = exchange_i32(packT, packB, lax.rem(g - g0, 2), jnp.add)

        cl_t = (ut | lax.shift_left(iota, s)) ^ MSB
        cl_b = (ub | lax.shift_left(iota, s) | low) ^ MSB
        decT = ((totT >= K_SEL) | (cl_t <= ktau_t)) & (iota >= 1)
        decB = ((totB >= K_SEL) | (cl_b >= ktau_b)) & (iota <= 14)
        sel = popcnt(decT)
        jstar = _splat(15) - popcnt(decB)
        return ut | lax.shift_left(sel, s), ub | lax.shift_left(jstar, s)

    ut, ub = lax.fori_loop(g0, 8, group, (ut0, ub0))
    ts_t = ut ^ MSB
    ts_b = ub ^ MSB
    x_t = inv_val(ts_t)
    x_b = inv_val(ts_b)

    def fin_TB(i, acc):
        svT, cvT, svB, cvB = acc
        xtv = bT[pl.ds(i * 16, 16)]
        xbv = bB[pl.ds(i * 16, 16)]
        mT = xtv > x_t
        mB = xbv < x_b
        return (svT + jnp.where(mT, xtv, zero_f), cvT + popcnt(mT),
                svB + jnp.where(mB, xbv, zero_f), cvB + popcnt(mB))

    sT, cT1, sB, cB1 = lax.fori_loop(
        0, tripJ, fin_TB, (zero_f, zero_i, zero_f, zero_i))

    stage_i[pl.ds(0, 16)] = cT1
    stage_i[pl.ds(16, 16)] = cB1
    stage_i[pl.ds(32, 16)] = plsc.bitcast(sT, jnp.int32)
    stage_i[pl.ds(48, 16)] = plsc.bitcast(sB, jnp.int32)
    pltpu.sync_copy(stage_i, sh_w.at[pl.ds(sid * 64, 64)])
    plsc.subcore_barrier()
    pltpu.sync_copy(sh_w, gbuf_i)
    cgt = gbuf_i[pl.ds(0, 16)]
    clt = gbuf_i[pl.ds(16, 16)]
    gs1 = plsc.bitcast(gbuf_i[pl.ds(32, 16)], jnp.float32)
    gs2 = plsc.bitcast(gbuf_i[pl.ds(48, 16)], jnp.float32)
    for w in range(1, NS):
        cgt = cgt + gbuf_i[pl.ds(w * 64, 16)]
        clt = clt + gbuf_i[pl.ds(w * 64 + 16, 16)]
        gs1 = gs1 + plsc.bitcast(gbuf_i[pl.ds(w * 64 + 32, 16)], jnp.float32)
        gs2 = gs2 + plsc.bitcast(gbuf_i[pl.ds(w * 64 + 48, 16)], jnp.float32)
    s_gt = jnp.sum(gs1)
    s_lt = jnp.sum(gs2)

    rem_t = (_splat(K_SEL) - cgt).astype(jnp.float32)
    rem_b = (_splat(K_SEL) - clt).astype(jnp.float32)
    s_top = _splat(s_gt, jnp.float32) + rem_t * x_t
    s_bot = _splat(s_lt, jnp.float32) + rem_b * x_b

    res = 2.0 * (s_top - s_bot) - (x_max - x_min)
    outv[...] = res

    @pl.when(jnp.logical_and(cid == 0, sid == 0))
    def _():
        pltpu.sync_copy(outv, out_hbm)


_toploss_sc = pl.kernel(
    _toploss_body,
    out_type=jax.ShapeDtypeStruct((16,), jnp.float32),
    mesh=_mesh,
    compiler_params=pltpu.CompilerParams(needs_layout_passes=False),
    scratch_types=_SCRATCH,
)


def kernel(beta, ground):
    del ground
    out = _toploss_sc(beta.reshape(-1))
    return out[0]


# speedup vs baseline: 1.0124x; 1.0124x over previous
import jax
import jax.numpy as jnp
from jax import lax
from jax.experimental import pallas as pl
from jax.experimental.pallas import tpu as pltpu
from jax.experimental.pallas import tpu_sc as plsc

N = 512 * 512
K_SEL = 1024
NS = 16
CH = N // NS
NV = CH // 16
MSB = -0x80000000

_mesh = plsc.VectorSubcoreMesh(
    core_axis_name="c", subcore_axis_name="s", num_cores=1, num_subcores=NS)


def _splat(v, dtype=jnp.int32):
    return jnp.full((16,), v, dtype=dtype)


_SCRATCH = [
    pltpu.VMEM((CH,), jnp.float32),
    pltpu.VMEM((CH + 16,), jnp.float32),
    pltpu.VMEM((CH + 16,), jnp.float32),
    pltpu.VMEM((64,), jnp.int32),
    pltpu.VMEM((NS * 64,), jnp.int32),
    pltpu.VMEM((16,), jnp.float32),
    pltpu.VMEM((32,), jnp.int32),
    pltpu.VMEM_SHARED((2 * NS * 32,), jnp.int32),
    pltpu.VMEM_SHARED((NS * 64,), jnp.int32),
]


def _toploss_body(x_hbm, out_hbm, xv, bT, bB,
                  stage_i, gbuf_i, outv, nbuf, sh_i, sh_w):
    cid = lax.axis_index("c")
    sid = lax.axis_index("s")
    base = sid * CH
    iota = lax.iota(jnp.int32, 16)
    zero_i = _splat(0)
    zero_f = _splat(0.0, jnp.float32)

    def popcnt(m):
        return plsc.all_reduce_population_count(m)

    def exchange_i32(lo_vec, hi_vec, parity, red):
        stage_i[pl.ds(0, 16)] = lo_vec
        stage_i[pl.ds(16, 16)] = hi_vec
        pltpu.sync_copy(stage_i.at[pl.ds(0, 32)],
                        sh_i.at[pl.ds(parity * (NS * 32) + sid * 32, 32)])
        plsc.subcore_barrier()
        pltpu.sync_copy(sh_i.at[pl.ds(parity * (NS * 32), NS * 32)],
                        gbuf_i.at[pl.ds(0, NS * 32)])
        lo = gbuf_i[pl.ds(0, 16)]
        hi = gbuf_i[pl.ds(16, 16)]
        for w in range(1, NS):
            lo = red(lo, gbuf_i[pl.ds(w * 32, 16)])
            hi = red(hi, gbuf_i[pl.ds(w * 32 + 16, 16)])
        return lo, hi

    def skey_of(xvec):
        b = plsc.bitcast(xvec, jnp.int32)
        return jnp.where(b >= 0, b, MSB - b)

    def inv_val(kvv):
        bits = jnp.where(kvv >= 0, kvv, MSB - kvv)
        return plsc.bitcast(bits, jnp.float32)

    pltpu.sync_copy(x_hbm.at[pl.ds(base, CH)], xv)

    @plsc.parallel_loop(0, NV, unroll=8,
                        carry=(_splat(-jnp.inf, jnp.float32),
                               _splat(jnp.inf, jnp.float32)))
    def _minmax(i, carry):
        mxf, mnf = carry
        xvec = xv[pl.ds(i * 16, 16)]
        return jnp.maximum(mxf, xvec), jnp.minimum(mnf, xvec)

    mxf, mnf = _minmax
    mxg, mng_inv = exchange_i32(skey_of(mxf), ~skey_of(mnf), 0, jnp.maximum)
    mng = ~mng_inv
    kmax_v = _splat(jnp.max(mxg))
    kmin_v = _splat(jnp.min(mng))
    x_max = inv_val(kmax_v)
    x_min = inv_val(kmin_v)

    rng_v = (x_max - x_min) * (1.0 / 160.0)
    tau_t_try = x_max - rng_v
    tau_b_try = x_min + rng_v

    def compact_pass(tt, tb):
        @plsc.parallel_loop(0, NV, unroll=8,
                            carry=(_splat(-1), _splat(-1)))
        def _comp(i, carry):
            offTm1, offBm1 = carry
            xvec = xv[pl.ds(i * 16, 16)]
            mT = xvec >= tt
            mB = xvec <= tb
            plsc.store_scatter(
                bT, [offTm1 + plsc.cumsum(mT.astype(jnp.int32))],
                xvec, mask=mT)
            plsc.store_scatter(
                bB, [offBm1 + plsc.cumsum(mB.astype(jnp.int32))],
                xvec, mask=mB)
            return offTm1 + popcnt(mT), offBm1 + popcnt(mB)

        om1T, om1B = _comp
        return om1T + 1, om1B + 1

    offT, offB = compact_pass(tau_t_try, tau_b_try)
    cT, cB = exchange_i32(offT, offB, 1, jnp.add)
    okT = cT >= K_SEL
    okB = cB >= K_SEL
    tau_t = jnp.where(okT, tau_t_try, x_min)
    tau_b = jnp.where(okB, tau_b_try, x_max)
    nbuf[pl.ds(0, 16)] = offT
    nbuf[pl.ds(16, 16)] = offB
    redo = jnp.max(jnp.where(okT & okB, zero_i, _splat(1)))

    @pl.when(redo > 0)
    def _():
        oT, oB = compact_pass(tau_t, tau_b)
        nbuf[pl.ds(0, 16)] = oT
        nbuf[pl.ds(16, 16)] = oB

    nT_v = nbuf[pl.ds(0, 16)]
    nB_v = nbuf[pl.ds(16, 16)]
    tripT = (jnp.max(nT_v) + 15) // 16
    tripB = (jnp.max(nB_v) + 15) // 16
    plsc.store_scatter(bT, [nT_v + iota], _splat(-jnp.inf, jnp.float32))
    plsc.store_scatter(bB, [nB_v + iota], _splat(jnp.inf, jnp.float32))

    ktau_t = skey_of(tau_t)
    ktau_b = skey_of(tau_b)

    def ehigh(diff):
        f = diff.astype(jnp.float32)
        e = lax.shift_right_logical(plsc.bitcast(f, jnp.int32), 23) - 127
        e = jnp.where(diff == 0, _splat(-1000), e)
        return jnp.where(diff < 0, _splat(31), e)

    e_joint = jnp.maximum(ehigh(ktau_t ^ kmax_v), ehigh(ktau_b ^ kmin_v))
    g0_v = jnp.clip((_splat(31) - e_joint) // 4, 0, 8)
    nclear = _splat(32) - 4 * g0_v
    lowmask = jnp.where(g0_v == 0, _splat(-1),
                        lax.shift_left(_splat(1), nclear) - 1)
    ut0 = ((kmax_v ^ MSB) & ~lowmask)
    ub0 = ((kmin_v ^ MSB) & ~lowmask)
    g0 = jnp.max(g0_v)

    def group(g, carry):
        ut, ub = carry
        s = 28 - 4 * g
        low = lax.shift_left(jnp.int32(1), s) - 1
        cand_ts = [inv_val((ut | lax.shift_left(_splat(j), s)) ^ MSB)
                   for j in range(1, 16)]
        cand_bs = [inv_val((ub | lax.shift_left(_splat(j), s) | low) ^ MSB)
                   for j in range(15)]

        def cnt_T(i, accs):
            xfv = bT[pl.ds(i * 16, 16)]
            return tuple(a + popcnt(xfv >= c)
                         for a, c in zip(accs, cand_ts))

        def cnt_B(i, accs):
            xfv = bB[pl.ds(i * 16, 16)]
            return tuple(a + popcnt(xfv <= c)
                         for a, c in zip(accs, cand_bs))

        accT = lax.fori_loop(0, tripT, cnt_T, (zero_i,) * 15)
        accB = lax.fori_loop(0, tripB, cnt_B, (zero_i,) * 15)
        packT = zero_i
        for j, a in enumerate(accT):
            packT = packT + jnp.where(iota == j + 1, a, zero_i)
        packB = zero_i
        for j, a in enumerate(accB):
            packB = packB + jnp.where(iota == j, a, zero_i)
        totT, totB = exchange_i32(packT, packB, lax.rem(g - g0, 2), jnp.add)

        cl_t = (ut | lax.shift_left(iota, s)) ^ MSB
        cl_b = (ub | lax.shift_left(iota, s) | low) ^ MSB
        decT = ((totT >= K_SEL) | (cl_t <= ktau_t)) & (iota >= 1)
        decB = ((totB >= K_SEL) | (cl_b >= ktau_b)) & (iota <= 14)
        sel = popcnt(decT)
        jstar = _splat(15) - popcnt(decB)
        return ut | lax.shift_left(sel, s), ub | lax.shift_left(jstar, s)

    ut, ub = lax.fori_loop(g0, 8, group, (ut0, ub0))
    ts_t = ut ^ MSB
    ts_b = ub ^ MSB
    x_t = inv_val(ts_t)
    x_b = inv_val(ts_b)

    def fin_T(i, acc):
        sv, cv = acc
        xfv = bT[pl.ds(i * 16, 16)]
        m = xfv > x_t
        return sv + jnp.where(m, xfv, zero_f), cv + popcnt(m)

    def fin_B(i, acc):
        sv, cv = acc
        xfv = bB[pl.ds(i * 16, 16)]
        m = xfv < x_b
        return sv + jnp.where(m, xfv, zero_f), cv + popcnt(m)

    sT, cT1 = lax.fori_loop(0, tripT, fin_T, (zero_f, zero_i))
    sB, cB1 = lax.fori_loop(0, tripB, fin_B, (zero_f, zero_i))

    stage_i[pl.ds(0, 16)] = cT1
    stage_i[pl.ds(16, 16)] = cB1
    stage_i[pl.ds(32, 16)] = plsc.bitcast(sT, jnp.int32)
    stage_i[pl.ds(48, 16)] = plsc.bitcast(sB, jnp.int32)
    pltpu.sync_copy(stage_i, sh_w.at[pl.ds(sid * 64, 64)])
    plsc.subcore_barrier()
    pltpu.sync_copy(sh_w, gbuf_i)
    cgt = gbuf_i[pl.ds(0, 16)]
    clt = gbuf_i[pl.ds(16, 16)]
    gs1 = plsc.bitcast(gbuf_i[pl.ds(32, 16)], jnp.float32)
    gs2 = plsc.bitcast(gbuf_i[pl.ds(48, 16)], jnp.float32)
    for w in range(1, NS):
        cgt = cgt + gbuf_i[pl.ds(w * 64, 16)]
        clt = clt + gbuf_i[pl.ds(w * 64 + 16, 16)]
        gs1 = gs1 + plsc.bitcast(gbuf_i[pl.ds(w * 64 + 32, 16)], jnp.float32)
        gs2 = gs2 + plsc.bitcast(gbuf_i[pl.ds(w * 64 + 48, 16)], jnp.float32)
    s_gt = jnp.sum(gs1)
    s_lt = jnp.sum(gs2)

    rem_t = (_splat(K_SEL) - cgt).astype(jnp.float32)
    rem_b = (_splat(K_SEL) - clt).astype(jnp.float32)
    s_top = _splat(s_gt, jnp.float32) + rem_t * x_t
    s_bot = _splat(s_lt, jnp.float32) + rem_b * x_b

    res = 2.0 * (s_top - s_bot) - (x_max - x_min)
    outv[...] = res

    @pl.when(jnp.logical_and(cid == 0, sid == 0))
    def _():
        pltpu.sync_copy(outv, out_hbm)


_toploss_sc = pl.kernel(
    _toploss_body,
    out_type=jax.ShapeDtypeStruct((16,), jnp.float32),
    mesh=_mesh,
    compiler_params=pltpu.CompilerParams(needs_layout_passes=False),
    scratch_types=_SCRATCH,
)


def kernel(beta, ground):
    del ground
    out = _toploss_sc(beta.reshape(-1))
    return out[0]
